# Initial kernel scaffold; baseline (speedup 1.0000x reference)
import functools

import jax
import jax.numpy as jnp
from jax import lax
from jax.experimental import pallas as pl
from jax.experimental.pallas import tpu as pltpu
from jax.experimental.pallas import tpu_sc as plsc

B = 1024
P = 100000
C = 100
K = 64

NC = 2
NS = 16
NW = NC * NS
ROWS_PER_W = B // NW

WIN = 4000
NWIN = P // WIN
GRP = 10
NGRP = (WIN // 16) // GRP
CT = 256
BUF = CT + WIN

_I32_MIN = jnp.int32(-(2**31))


def _f2key(v):
    s = plsc.bitcast(v, jnp.int32)
    m = (s >> 31) | _I32_MIN
    return plsc.bitcast(s ^ m, jnp.uint32)


def _key2f(k):
    ki = plsc.bitcast(k, jnp.int32)
    neg = ki >= 0
    fi = jnp.where(neg, ~ki, ki ^ _I32_MIN)
    return plsc.bitcast(fi, jnp.float32)


def _splat(x, dtype=jnp.int32):
    return jnp.full((16,), x, dtype)


def _scal(v):
    return lax.reduce_max(v, axes=(0,))


def _labels_tc(oh):
    rows = 1000
    grid = P // rows

    def body(oh_ref, out_ref):
        cls = lax.broadcasted_iota(jnp.float32, (1, C), 1)
        s = jnp.sum(oh_ref[...] * cls, axis=1)
        out_ref[0, 0, :] = s.astype(jnp.int32)

    out = pl.pallas_call(
        body,
        grid=(grid,),
        in_specs=[pl.BlockSpec((rows, C), lambda i: (i, 0))],
        out_specs=pl.BlockSpec((1, 1, rows), lambda i: (i, 0, 0)),
        out_shape=jax.ShapeDtypeStruct((grid, 1, rows), jnp.int32),
    )(oh)
    return out.reshape(P)


def _sc_knnc(x, labels):
    mesh = plsc.VectorSubcoreMesh(
        core_axis_name="c", subcore_axis_name="s", num_cores=NC, num_subcores=NS
    )

    @functools.partial(
        pl.kernel,
        out_type=jax.ShapeDtypeStruct((B,), jnp.int32),
        mesh=mesh,
        scratch_types=[
            pltpu.VMEM((P,), jnp.int32),
            pltpu.VMEM((2, WIN), jnp.float32),
            pltpu.VMEM((BUF,), jnp.uint32),
            pltpu.VMEM((BUF,), jnp.int32),
            pltpu.VMEM((256,), jnp.int32),
            pltpu.VMEM((C * 16,), jnp.int32),
            pltpu.VMEM((ROWS_PER_W,), jnp.int32),
            pltpu.SemaphoreType.DMA,
            pltpu.SemaphoreType.DMA,
        ],
    )
    def sc_kernel(x_hbm, lab_hbm, out_hbm, labels_v, win_v, keys_v, labs_v,
                  hist_v, vote_v, out_v, sem0, sem1):
        wid = lax.axis_index("s") * NC + lax.axis_index("c")
        row0 = wid * ROWS_PER_W
        lane = lax.iota(jnp.int32, 16)
        zeros16 = _splat(0)
        ones16 = _splat(1)

        pltpu.sync_copy(lab_hbm, labels_v)

        def compact(cnt):
            nv = (cnt + 15) // 16
            cnt_s = _splat(cnt)

            def radix_pass(ppass, st):
                prefix, pmask, target, n_lt = st
                shift = (28 - 4 * ppass).astype(jnp.uint32)
                shift_v = _splat(shift, jnp.uint32)
                for i in range(16):
                    hist_v[pl.ds(i * 16, 16)] = zeros16

                def scan(i, _):
                    kv = keys_v[pl.ds(i * 16, 16)]
                    valid = (lane + i * 16) < cnt_s
                    match = (kv & _splat(pmask, jnp.uint32)) == _splat(prefix, jnp.uint32)
                    ok = valid & match
                    digit = ((kv >> shift_v) & _splat(15, jnp.uint32)).astype(jnp.int32)
                    idxv = lane * 16 + digit
                    plsc.addupdate_scatter(hist_v, [idxv], jnp.where(ok, 1, 0))
                    return 0

                lax.fori_loop(0, nv, scan, 0)
                totals = zeros16
                for r in range(16):
                    totals = totals + hist_v[pl.ds(r * 16, 16)]
                cum = plsc.cumsum(totals)
                dstar = plsc.all_reduce_ffs(cum >= _splat(target))
                below = lax.reduce_sum(jnp.where(lane < dstar, totals, 0), axes=(0,))
                d_s = _scal(dstar).astype(jnp.uint32)
                prefix = prefix | (d_s << shift)
                pmask = pmask | (jnp.uint32(15) << shift)
                return (prefix, pmask, target - below, n_lt + below)

            prefix, _, m, n_lt = lax.fori_loop(
                0, 8, radix_pass,
                (jnp.uint32(0), jnp.uint32(0), jnp.int32(K), jnp.int32(0)))
            v64 = _splat(prefix, jnp.uint32)
            m_s = _splat(m)

            def rewrite(i, st):
                wofs, eqc = st
                kv = keys_v[pl.ds(i * 16, 16)]
                lb = labs_v[pl.ds(i * 16, 16)]
                valid = (lane + i * 16) < cnt_s
                lt = (kv < v64) & valid
                eq = (kv == v64) & valid
                eqi = jnp.where(eq, 1, 0)
                eqrank = _splat(eqc) + plsc.cumsum(eqi) - eqi
                keep = lt | (eq & (eqrank < m_s))
                ki = jnp.where(keep, 1, 0)
                pos = _splat(wofs) + plsc.cumsum(ki) - ki
                plsc.store_scatter(keys_v, [pos], kv, mask=keep)
                plsc.store_scatter(labs_v, [pos], lb, mask=keep)
                wofs = wofs + _scal(plsc.all_reduce_population_count(keep))
                eqc = eqc + _scal(plsc.all_reduce_population_count(eq))
                return (wofs, eqc)

            lax.fori_loop(0, nv, rewrite, (jnp.int32(0), jnp.int32(0)))
            return jnp.int32(K), _key2f(v64)

        def append_vregs(base_off, gbase, n, cnt, tau_vec, pbuf):
            def vbody(j, cnt):
                off = base_off + j * 16
                v = win_v[pbuf, pl.ds(off, 16)]
                msk = v < tau_vec
                key = _f2key(v)
                gidx = gbase + j * 16 + lane
                lb = plsc.load_gather(labels_v, [gidx])
                mi = jnp.where(msk, 1, 0)
                pos = _splat(cnt) + plsc.cumsum(mi) - mi
                plsc.store_scatter(keys_v, [pos], key, mask=msk)
                plsc.store_scatter(labs_v, [pos], lb, mask=msk)
                return cnt + _scal(plsc.all_reduce_population_count(msk))

            return lax.fori_loop(0, n, vbody, cnt)

        def row_body(r, _):
            row = row0 + r
            pltpu.async_copy(x_hbm.at[row, pl.ds(0, WIN)], win_v.at[0], sem0).wait()

            inf16 = _splat(jnp.inf, jnp.float32)
            cnt = append_vregs(0, 0, GRP, jnp.int32(0), inf16, 0)
            cnt, tau_vec = compact(cnt)

            def win_body(w, carry):
                cnt, tau_vec = carry
                pbuf = w & 1

                @pl.when(w > 0)
                def _():
                    pltpu.make_async_copy(
                        x_hbm.at[row, pl.ds(w * WIN, WIN)], win_v.at[pbuf],
                        sem0).wait()

                @pl.when(w + 1 < NWIN)
                def _():
                    pltpu.async_copy(
                        x_hbm.at[row, pl.ds((w + 1) * WIN, WIN)],
                        win_v.at[1 - pbuf], sem0)

                cnt, tau_vec = lax.cond(
                    cnt >= CT, lambda c, t: compact(c), lambda c, t: (c, t),
                    cnt, tau_vec)

                def group(g, cnt):
                    base = g * GRP * 16
                    anyhit = win_v[pbuf, pl.ds(base, 16)] < tau_vec
                    for j in range(1, GRP):
                        anyhit = anyhit | (win_v[pbuf, pl.ds(base + j * 16, 16)] < tau_vec)
                    nhit = _scal(plsc.all_reduce_population_count(anyhit))
                    return lax.cond(
                        nhit > 0,
                        lambda c: append_vregs(base, w * WIN + base, GRP, c,
                                               tau_vec, pbuf),
                        lambda c: c, cnt)

                for g in range(NGRP):
                    if g == 0:
                        cnt = lax.cond(w > 0, lambda c: group(0, c),
                                       lambda c: c, cnt)
                    else:
                        cnt = group(g, cnt)
                return (cnt, tau_vec)

            cnt, tau_vec = lax.fori_loop(0, NWIN, win_body, (cnt, tau_vec))
            cnt, tau_vec = compact(cnt)

            for i in range(C):
                vote_v[pl.ds(i * 16, 16)] = zeros16
            for j in range(K // 16):
                lb = labs_v[pl.ds(j * 16, 16)]
                plsc.addupdate_scatter(vote_v, [lb * 16 + lane], ones16)

            def argmax_body(c, st):
                best, bc = st
                tot = lax.reduce_sum(vote_v[pl.ds(c * 16, 16)], axes=(0,))
                better = tot > best
                return (jnp.where(better, tot, best), jnp.where(better, c, bc))

            _, bc = lax.fori_loop(0, C, argmax_body, (jnp.int32(-1), jnp.int32(0)))
            plsc.store_scatter(out_v, [_splat(r)], _splat(bc), mask=lane == 0)
            return 0

        lax.fori_loop(0, ROWS_PER_W, row_body, 0)
        pltpu.sync_copy(out_v, out_hbm.at[pl.ds(row0, ROWS_PER_W)])

    return sc_kernel(x, labels)


def kernel(x, oh_prototype_labels):
    labels = _labels_tc(oh_prototype_labels)
    return _sc_knnc(x, labels)


# speedup vs baseline: 4.9153x; 4.9153x over previous
import functools

import jax
import jax.numpy as jnp
import numpy as np
from jax import lax
from jax.experimental import pallas as pl
from jax.experimental.pallas import tpu as pltpu
from jax.experimental.pallas import tpu_sc as plsc

B = 1024
P = 100000
C = 100
K = 64

NC = 2
NS = 16
NW = NC * NS
ROWS_PER_W = B // NW
NRG = ROWS_PER_W // 8

W = 640
NWF = P // W
TAIL = P - NWF * W
NWIN = NWF + 1
GRP = 8
NGRP = (W // 16) // GRP
GRP_T = 10
NGRP_T = (TAIL // 16) // GRP_T
CT = 256
BUF = CT + W

_I32_MAX = np.int32(2**31 - 1)


def _f2key(v):
    s = lax.bitcast_convert_type(v, jnp.int32)
    return s ^ ((s >> 31) & _I32_MAX)


def _key2f(k):
    return lax.bitcast_convert_type(k ^ ((k >> 31) & _I32_MAX), jnp.float32)


def _splat(x, dtype=jnp.int32):
    return jnp.full((16,), x, dtype)


def _scal(v):
    return lax.reduce_max(v, axes=(0,))


def _labels_tc(oh):
    rows = 1000
    grid = P // rows

    def body(oh_ref, out_ref):
        cls = lax.broadcasted_iota(jnp.int32, (1, C), 1).astype(jnp.float32)
        s = jnp.sum(oh_ref[...] * cls, axis=1)
        out_ref[0, 0, :] = s.astype(jnp.int32)

    out = pl.pallas_call(
        body,
        grid=(grid,),
        in_specs=[pl.BlockSpec((rows, C), lambda i: (i, 0))],
        out_specs=pl.BlockSpec((1, 1, rows), lambda i: (i, 0, 0)),
        out_shape=jax.ShapeDtypeStruct((grid, 1, rows), jnp.int32),
    )(oh)
    return out.reshape(P)


def _sc_knnc(x, x_tail, labels):
    mesh = plsc.VectorSubcoreMesh(
        core_axis_name="c", subcore_axis_name="s", num_cores=NC, num_subcores=NS
    )

    @functools.partial(
        pl.kernel,
        out_type=jax.ShapeDtypeStruct((B,), jnp.int32),
        mesh=mesh,
        compiler_params=pltpu.CompilerParams(needs_layout_passes=False),
        scratch_types=[
            pltpu.VMEM((P,), jnp.int32),
            pltpu.VMEM((2, 8, W), jnp.float32),
            pltpu.VMEM((2, 8, TAIL), jnp.float32),
            pltpu.VMEM((8, BUF), jnp.int32),
            pltpu.VMEM((8, BUF), jnp.int32),
            pltpu.VMEM((256,), jnp.int32),
            pltpu.VMEM((C * 16,), jnp.int32),
            pltpu.VMEM((ROWS_PER_W,), jnp.int32),
            pltpu.SMEM((8,), jnp.int32),
            pltpu.SMEM((8,), jnp.float32),
            pltpu.SemaphoreType.DMA,
        ],
    )
    def sc_kernel(x_hbm, xt_hbm, lab_hbm, out_hbm, labels_v, win_v, win_t,
                  keys_v, labs_v, hist_v, vote_v, out_v, cnt_s8, tau_s8, sem0):
        wid = lax.axis_index("s") * NC + lax.axis_index("c")
        row0 = wid * ROWS_PER_W
        lane = lax.iota(jnp.int32, 16)
        zeros16 = _splat(0)
        ones16 = _splat(1)

        pltpu.sync_copy(lab_hbm, labels_v)

        def compact(r8, cnt):
            nv = (cnt + 15) // 16
            cnt_v = _splat(cnt)
            r8_v = _splat(r8)

            def radix_pass(ppass, st):
                prefix, pmask, target, n_lt = st
                shift = 28 - 4 * ppass
                shift_v = _splat(shift)
                oflip = jnp.where(ppass == 0, 8, 0)
                for i in range(16):
                    hist_v[pl.ds(i * 16, 16)] = zeros16

                def scan(i, _):
                    kv = keys_v[r8, pl.ds(i * 16, 16)]
                    valid = (lane + i * 16) < cnt_v
                    match = (kv & _splat(pmask)) == _splat(prefix)
                    ok = valid & match
                    od = ((kv >> shift_v) & _splat(15)) ^ _splat(oflip)
                    idxv = lane * 16 + od
                    plsc.addupdate_scatter(hist_v, [idxv], jnp.where(ok, 1, 0))
                    return 0

                lax.fori_loop(0, nv, scan, 0)
                totals = zeros16
                for r in range(16):
                    totals = totals + hist_v[pl.ds(r * 16, 16)]
                cum = plsc.cumsum(totals)
                dstar = plsc.all_reduce_ffs(cum >= _splat(target))
                below = lax.reduce_sum(jnp.where(lane < dstar, totals, 0), axes=(0,))
                d_s = _scal(dstar) ^ oflip
                prefix = prefix | (d_s << shift)
                pmask = pmask | (15 << shift)
                return (prefix, pmask, target - below, n_lt + below)

            prefix, _, m, n_lt = lax.fori_loop(
                0, 8, radix_pass,
                (jnp.int32(0), jnp.int32(0), jnp.int32(K), jnp.int32(0)))
            v64 = _splat(prefix)
            m_v = _splat(m)

            def rewrite(i, st):
                wofs, eqc = st
                kv = keys_v[r8, pl.ds(i * 16, 16)]
                lb = labs_v[r8, pl.ds(i * 16, 16)]
                valid = (lane + i * 16) < cnt_v
                lt = (kv < v64) & valid
                eq = (kv == v64) & valid
                eqi = jnp.where(eq, 1, 0)
                eqrank = _splat(eqc) + plsc.cumsum(eqi) - eqi
                keep = lt | (eq & (eqrank < m_v))
                ki = jnp.where(keep, 1, 0)
                pos = _splat(wofs) + plsc.cumsum(ki) - ki
                plsc.store_scatter(keys_v, [r8_v, pos], kv, mask=keep)
                plsc.store_scatter(labs_v, [r8_v, pos], lb, mask=keep)
                wofs = wofs + _scal(plsc.all_reduce_population_count(keep))
                eqc = eqc + _scal(plsc.all_reduce_population_count(eq))
                return (wofs, eqc)

            lax.fori_loop(0, nv, rewrite, (jnp.int32(0), jnp.int32(0)))
            tau = _scal(_key2f(v64))
            return jnp.int32(K), tau

        def append_vregs(wref, pbuf, r8, base_off, gbase, n, cnt, tau_vec):
            def vbody(j, cnt):
                off = base_off + j * 16
                v = wref[pbuf, r8, pl.ds(off, 16)]
                msk = v < tau_vec
                key = _f2key(v)
                gidx = gbase + j * 16 + lane
                lb = plsc.load_gather(labels_v, [gidx])
                mi = jnp.where(msk, 1, 0)
                pos = _splat(cnt) + plsc.cumsum(mi) - mi
                r8_v = _splat(r8)
                plsc.store_scatter(keys_v, [r8_v, pos], key, mask=msk)
                plsc.store_scatter(labs_v, [r8_v, pos], lb, mask=msk)
                return cnt + _scal(plsc.all_reduce_population_count(msk))

            return lax.fori_loop(0, n, vbody, cnt)

        def groups_loop(wref, pbuf, r8, w, glo, ngrp, grp, cnt, tau_vec):
            def gbody(g, cnt):
                base = g * grp * 16
                anyhit = wref[pbuf, r8, pl.ds(base, 16)] < tau_vec
                for j in range(1, grp):
                    anyhit = anyhit | (
                        wref[pbuf, r8, pl.ds(base + j * 16, 16)] < tau_vec)
                nhit = _scal(plsc.all_reduce_population_count(anyhit))
                return lax.cond(
                    nhit > 0,
                    lambda c: append_vregs(wref, pbuf, r8, base, w * W + base,
                                           grp, c, tau_vec),
                    lambda c: c, cnt)

            return lax.fori_loop(glo, ngrp, gbody, cnt)

        def rg_body(rg, _):
            rgbase = pl.multiple_of(row0 + rg * 8, 8)

            pltpu.async_copy(
                x_hbm.at[pl.ds(rgbase, 8), pl.ds(0, W)], win_v.at[0],
                sem0).wait()

            def prologue(r8, _):
                inf16 = _splat(jnp.inf, jnp.float32)
                cnt = append_vregs(win_v, 0, r8, 0, 0, GRP, jnp.int32(0), inf16)
                cnt, tau = compact(r8, cnt)
                cnt_s8[r8] = cnt
                tau_s8[r8] = tau
                return 0

            lax.fori_loop(0, 8, prologue, 0)

            def win_body(w, _):
                pbuf = w & 1

                @pl.when((w > 0) & (w < NWF))
                def _():
                    cb = pl.multiple_of(w * W, 128)
                    pltpu.make_async_copy(
                        x_hbm.at[pl.ds(rgbase, 8), pl.ds(cb, W)],
                        win_v.at[pbuf], sem0).wait()

                @pl.when(w == NWF)
                def _():
                    pltpu.make_async_copy(
                        xt_hbm.at[pl.ds(rgbase, 8)],
                        win_t.at[pbuf], sem0).wait()

                @pl.when(w + 1 < NWF)
                def _():
                    cb = pl.multiple_of((w + 1) * W, 128)
                    pltpu.async_copy(
                        x_hbm.at[pl.ds(rgbase, 8), pl.ds(cb, W)],
                        win_v.at[1 - pbuf], sem0)

                @pl.when(w + 1 == NWF)
                def _():
                    pltpu.async_copy(
                        xt_hbm.at[pl.ds(rgbase, 8)],
                        win_t.at[1 - pbuf], sem0)

                def per_row(r8, _):
                    cnt = cnt_s8[r8]
                    tau = tau_s8[r8]
                    cnt, tau = lax.cond(cnt >= CT,
                                        lambda c, t: compact(r8, c),
                                        lambda c, t: (c, t), cnt, tau)
                    tau_vec = _splat(tau, jnp.float32)
                    glo = jnp.where(w == 0, 1, 0)
                    cnt = lax.cond(
                        w < NWF,
                        lambda c: groups_loop(win_v, pbuf, r8, w, glo, NGRP,
                                              GRP, c, tau_vec),
                        lambda c: groups_loop(win_t, pbuf, r8, w, 0, NGRP_T,
                                              GRP_T, c, tau_vec),
                        cnt)
                    cnt_s8[r8] = cnt
                    tau_s8[r8] = tau
                    return 0

                lax.fori_loop(0, 8, per_row, 0)
                return 0

            lax.fori_loop(0, NWIN, win_body, 0)

            def finalize(r8, _):
                cnt = cnt_s8[r8]
                cnt, tau = compact(r8, cnt)

                for i in range(C):
                    vote_v[pl.ds(i * 16, 16)] = zeros16
                for j in range(K // 16):
                    lb = labs_v[r8, pl.ds(j * 16, 16)]
                    plsc.addupdate_scatter(vote_v, [lb * 16 + lane], ones16)

                def argmax_body(c, st):
                    best, bc = st
                    tot = lax.reduce_sum(vote_v[pl.ds(c * 16, 16)], axes=(0,))
                    better = tot > best
                    return (jnp.where(better, tot, best),
                            jnp.where(better, c, bc))

                _, bc = lax.fori_loop(0, C, argmax_body,
                                      (jnp.int32(-1), jnp.int32(0)))
                plsc.store_scatter(out_v, [_splat(rg * 8 + r8)], _splat(bc),
                                   mask=lane == 0)
                return 0

            lax.fori_loop(0, 8, finalize, 0)
            return 0

        lax.fori_loop(0, NRG, rg_body, 0)
        pltpu.sync_copy(out_v, out_hbm.at[pl.ds(row0, ROWS_PER_W)])

    return sc_kernel(x, x_tail, labels)


def kernel(x, oh_prototype_labels):
    labels = _labels_tc(oh_prototype_labels)
    x_tail = lax.slice(x, (0, NWF * W), (B, P))
    return _sc_knnc(x, x_tail, labels)


# speedup vs baseline: 5.9236x; 1.2051x over previous
import functools

import jax
import jax.numpy as jnp
import numpy as np
from jax import lax
from jax.experimental import pallas as pl
from jax.experimental.pallas import tpu as pltpu
from jax.experimental.pallas import tpu_sc as plsc

B = 1024
P = 100000
C = 100
K = 64

NC = 2
NS = 16
NW = NC * NS
ROWS_PER_W = B // NW
NRG = ROWS_PER_W // 8

W = 640
NWF = P // W
TAIL = P - NWF * W
NWIN = NWF + 1
GRP = 8
NGRP = (W // 16) // GRP
GRP_T = 10
NGRP_T = (TAIL // 16) // GRP_T
CT = 256
BUF = CT + W

_I32_MAX = np.int32(2**31 - 1)


def _f2key(v):
    s = lax.bitcast_convert_type(v, jnp.int32)
    return s ^ ((s >> 31) & _I32_MAX)


def _key2f(k):
    return lax.bitcast_convert_type(k ^ ((k >> 31) & _I32_MAX), jnp.float32)


def _splat(x, dtype=jnp.int32):
    return jnp.full((16,), x, dtype)


def _scal(v):
    return lax.reduce_max(v, axes=(0,))


def _lane0(v):
    return lax.squeeze(lax.slice_in_dim(v, 0, 1), dimensions=(0,))


def _labels_tc(oh):
    rows = 1000
    grid = P // rows

    def body(oh_ref, out_ref):
        cls = lax.broadcasted_iota(jnp.int32, (1, C), 1).astype(jnp.float32)
        s = jnp.sum(oh_ref[...] * cls, axis=1)
        out_ref[0, 0, :] = s.astype(jnp.int32)

    out = pl.pallas_call(
        body,
        grid=(grid,),
        in_specs=[pl.BlockSpec((rows, C), lambda i: (i, 0))],
        out_specs=pl.BlockSpec((1, 1, rows), lambda i: (i, 0, 0)),
        out_shape=jax.ShapeDtypeStruct((grid, 1, rows), jnp.int32),
    )(oh)
    return out.reshape(P)


def _sc_knnc(x, x_tail, labels):
    mesh = plsc.VectorSubcoreMesh(
        core_axis_name="c", subcore_axis_name="s", num_cores=NC, num_subcores=NS
    )

    @functools.partial(
        pl.kernel,
        out_type=jax.ShapeDtypeStruct((B,), jnp.int32),
        mesh=mesh,
        compiler_params=pltpu.CompilerParams(needs_layout_passes=False),
        scratch_types=[
            pltpu.VMEM((P,), jnp.int32),
            pltpu.VMEM((2, 8, W), jnp.float32),
            pltpu.VMEM((2, 8, TAIL), jnp.float32),
            pltpu.VMEM((8, BUF), jnp.int32),
            pltpu.VMEM((8, BUF), jnp.int32),
            pltpu.VMEM((256,), jnp.int32),
            pltpu.VMEM((C * 16,), jnp.int32),
            pltpu.VMEM((ROWS_PER_W,), jnp.int32),
            pltpu.SMEM((8,), jnp.int32),
            pltpu.SMEM((8,), jnp.float32),
            pltpu.SemaphoreType.DMA,
        ],
    )
    def sc_kernel(x_hbm, xt_hbm, lab_hbm, out_hbm, labels_v, win_v, win_t,
                  keys_v, labs_v, hist_v, vote_v, out_v, cnt_s8, tau_s8, sem0):
        wid = lax.axis_index("s") * NC + lax.axis_index("c")
        row0 = wid * ROWS_PER_W
        lane = lax.iota(jnp.int32, 16)
        zeros16 = _splat(0)
        ones16 = _splat(1)

        pltpu.sync_copy(lab_hbm, labels_v)

        def compact(r8, cnt):
            nv = (cnt + 15) // 16
            cnt_v = _splat(cnt)

            def radix_pass(ppass, st):
                prefix, pmask, target, n_lt = st
                shift = 28 - 4 * ppass
                shift_v = _splat(shift)
                oflip = jnp.where(ppass == 0, 8, 0)
                for i in range(16):
                    hist_v[pl.ds(i * 16, 16)] = zeros16

                def scan(i, _):
                    kv = keys_v[r8, pl.ds(i * 16, 16)]
                    valid = (lane + i * 16) < cnt_v
                    match = (kv & _splat(pmask)) == _splat(prefix)
                    ok = valid & match
                    od = ((kv >> shift_v) & _splat(15)) ^ _splat(oflip)
                    idxv = lane * 16 + od
                    plsc.addupdate_scatter(hist_v, [idxv], jnp.where(ok, 1, 0))
                    return 0

                lax.fori_loop(0, nv, scan, 0)
                totals = zeros16
                for r in range(16):
                    totals = totals + hist_v[pl.ds(r * 16, 16)]
                cum = plsc.cumsum(totals)
                dstar = plsc.all_reduce_ffs(cum >= _splat(target))
                below = lax.reduce_sum(jnp.where(lane < dstar, totals, 0), axes=(0,))
                d_s = _lane0(dstar) ^ oflip
                prefix = prefix | (d_s << shift)
                pmask = pmask | (15 << shift)
                return (prefix, pmask, target - below, n_lt + below)

            prefix, _, m, n_lt = lax.fori_loop(
                0, 8, radix_pass,
                (jnp.int32(0), jnp.int32(0), jnp.int32(K), jnp.int32(0)))
            v64 = _splat(prefix)
            m_v = _splat(m)

            def rewrite(i, st):
                wofs, eqc = st
                kv = keys_v[r8, pl.ds(i * 16, 16)]
                lb = labs_v[r8, pl.ds(i * 16, 16)]
                valid = (lane + i * 16) < cnt_v
                lt = (kv < v64) & valid
                eq = (kv == v64) & valid
                eqi = jnp.where(eq, 1, 0)
                eqrank = _splat(eqc) + plsc.cumsum(eqi) - eqi
                keep = lt | (eq & (eqrank < m_v))
                plsc.store_compressed(keys_v.at[r8, pl.ds(wofs, 16)], kv,
                                      mask=keep)
                plsc.store_compressed(labs_v.at[r8, pl.ds(wofs, 16)], lb,
                                      mask=keep)
                wofs = wofs + _lane0(plsc.all_reduce_population_count(keep))
                eqc = eqc + _lane0(plsc.all_reduce_population_count(eq))
                return (wofs, eqc)

            lax.fori_loop(0, nv, rewrite, (jnp.int32(0), jnp.int32(0)))
            tau = _lane0(_key2f(v64))
            return jnp.int32(K), tau

        def append_vregs(wref, pbuf, r8, base_off, gbase, n, cnt, tau_vec):
            def vbody(j, cnt):
                off = base_off + j * 16
                v = wref[pbuf, r8, pl.ds(off, 16)]
                msk = v < tau_vec
                key = _f2key(v)
                gidx = gbase + j * 16 + lane
                lb = plsc.load_gather(labels_v, [gidx])
                plsc.store_compressed(keys_v.at[r8, pl.ds(cnt, 16)], key,
                                      mask=msk)
                plsc.store_compressed(labs_v.at[r8, pl.ds(cnt, 16)], lb,
                                      mask=msk)
                return cnt + _lane0(plsc.all_reduce_population_count(msk))

            return lax.fori_loop(0, n, vbody, cnt)

        def groups_loop(wref, pbuf, r8, w, skip0, ngrp, grp, cnt, tau_vec):
            for g in range(ngrp):
                base = g * grp * 16

                def gbody(cnt, base=base):
                    anyhit = wref[pbuf, r8, pl.ds(base, 16)] < tau_vec
                    for j in range(1, grp):
                        anyhit = anyhit | (
                            wref[pbuf, r8, pl.ds(base + j * 16, 16)] < tau_vec)
                    return lax.cond(
                        jnp.any(anyhit),
                        lambda c: append_vregs(wref, pbuf, r8, base,
                                               w * W + base, grp, c, tau_vec),
                        lambda c: c, cnt)

                if skip0 and g == 0:
                    cnt = lax.cond(w > 0, gbody, lambda c: c, cnt)
                else:
                    cnt = gbody(cnt)
            return cnt

        def rg_body(rg, _):
            rgbase = pl.multiple_of(row0 + rg * 8, 8)

            pltpu.async_copy(
                x_hbm.at[pl.ds(rgbase, 8), pl.ds(0, W)], win_v.at[0],
                sem0).wait()

            def prologue(r8, _):
                inf16 = _splat(jnp.inf, jnp.float32)
                cnt = append_vregs(win_v, 0, r8, 0, 0, GRP, jnp.int32(0), inf16)
                cnt, tau = compact(r8, cnt)
                cnt_s8[r8] = cnt
                tau_s8[r8] = tau
                return 0

            lax.fori_loop(0, 8, prologue, 0)

            def win_body(w, _):
                pbuf = w & 1

                @pl.when((w > 0) & (w < NWF))
                def _():
                    cb = pl.multiple_of(w * W, 128)
                    pltpu.make_async_copy(
                        x_hbm.at[pl.ds(rgbase, 8), pl.ds(cb, W)],
                        win_v.at[pbuf], sem0).wait()

                @pl.when(w == NWF)
                def _():
                    pltpu.make_async_copy(
                        xt_hbm.at[pl.ds(rgbase, 8)],
                        win_t.at[pbuf], sem0).wait()

                @pl.when(w + 1 < NWF)
                def _():
                    cb = pl.multiple_of((w + 1) * W, 128)
                    pltpu.async_copy(
                        x_hbm.at[pl.ds(rgbase, 8), pl.ds(cb, W)],
                        win_v.at[1 - pbuf], sem0)

                @pl.when(w + 1 == NWF)
                def _():
                    pltpu.async_copy(
                        xt_hbm.at[pl.ds(rgbase, 8)],
                        win_t.at[1 - pbuf], sem0)

                def per_row(r8, _):
                    cnt = cnt_s8[r8]
                    tau = tau_s8[r8]
                    cnt, tau = lax.cond(cnt >= CT,
                                        lambda c, t: compact(r8, c),
                                        lambda c, t: (c, t), cnt, tau)
                    tau_vec = _splat(tau, jnp.float32)
                    cnt = lax.cond(
                        w < NWF,
                        lambda c: groups_loop(win_v, pbuf, r8, w, True, NGRP,
                                              GRP, c, tau_vec),
                        lambda c: groups_loop(win_t, pbuf, r8, w, False,
                                              NGRP_T, GRP_T, c, tau_vec),
                        cnt)
                    cnt_s8[r8] = cnt
                    tau_s8[r8] = tau
                    return 0

                lax.fori_loop(0, 8, per_row, 0)
                return 0

            lax.fori_loop(0, NWIN, win_body, 0)

            def finalize(r8, _):
                cnt = cnt_s8[r8]
                cnt, tau = compact(r8, cnt)

                for i in range(C):
                    vote_v[pl.ds(i * 16, 16)] = zeros16
                for j in range(K // 16):
                    lb = labs_v[r8, pl.ds(j * 16, 16)]
                    plsc.addupdate_scatter(vote_v, [lb * 16 + lane], ones16)

                def argmax_body(c, st):
                    best, bc = st
                    tot = lax.reduce_sum(vote_v[pl.ds(c * 16, 16)], axes=(0,))
                    better = tot > best
                    return (jnp.where(better, tot, best),
                            jnp.where(better, c, bc))

                _, bc = lax.fori_loop(0, C, argmax_body,
                                      (jnp.int32(-1), jnp.int32(0)))
                plsc.store_scatter(out_v, [_splat(rg * 8 + r8)], _splat(bc),
                                   mask=lane == 0)
                return 0

            lax.fori_loop(0, 8, finalize, 0)
            return 0

        lax.fori_loop(0, NRG, rg_body, 0)
        pltpu.sync_copy(out_v, out_hbm.at[pl.ds(row0, ROWS_PER_W)])

    return sc_kernel(x, x_tail, labels)


def kernel(x, oh_prototype_labels):
    labels = _labels_tc(oh_prototype_labels)
    x_tail = lax.slice(x, (0, NWF * W), (B, P))
    return _sc_knnc(x, x_tail, labels)
